# Initial kernel scaffold; baseline (speedup 1.0000x reference)
import jax
import jax.numpy as jnp
from jax.experimental import pallas as pl


def kernel(idxs, positive_samples, negative_samples, U):
    raise NotImplementedError("write your pallas kernel here")


# speedup vs baseline: 1.2485x; 1.2485x over previous
import functools

import jax
import jax.numpy as jnp
from jax import lax
from jax.experimental import pallas as pl
from jax.experimental.pallas import tpu as pltpu
from jax.experimental.pallas import tpu_sc as plsc

_GATHER_DN = lax.GatherDimensionNumbers(
    offset_dims=(), collapsed_slice_dims=(0,), start_index_map=(0,))


def _permute(v, idx):
    return lax.gather(v, idx[:, None], _GATHER_DN, slice_sizes=(1,),
                      mode=lax.GatherScatterMode.PROMISE_IN_BOUNDS)


_BATCH = 16384
_D = 64
_MARGIN = 10.0
_NC = 2
_NS = 16
_NW = _NC * _NS
_ROWS_PER_W = _BATCH // _NW
_CHUNK = 128
_NCHUNK = _ROWS_PER_W // _CHUNK


def _sc_body(idx_hbm, pos_hbm, neg_hbm, u_hbm, out_hbm,
             idx_v, pos_v, neg_v, u_v, acc_v, sem_pos, sem_neg):
    wid = lax.axis_index("s") * _NC + lax.axis_index("c")

    pltpu.sync_copy(idx_hbm.at[pl.ds(wid * _NCHUNK, _NCHUNK)], idx_v)
    pltpu.sync_copy(u_hbm, u_v)

    u0 = u_v[0, pl.ds(0, 16)]
    u1 = u_v[0, pl.ds(16, 16)]
    u2 = u_v[0, pl.ds(32, 16)]
    u3 = u_v[0, pl.ds(48, 16)]
    lanes = lax.iota(jnp.int32, 16)
    lane15 = lanes == 15
    perms = [lanes ^ (1 << k) for k in range(4)]

    acc = jnp.zeros((16,), jnp.float32)
    for j in range(_NCHUNK):
        cp = pltpu.async_copy(pos_hbm.at[idx_v.at[j]], pos_v.at[j], sem_pos)
        cn = pltpu.async_copy(neg_hbm.at[idx_v.at[j]], neg_v.at[j], sem_neg)
        cp.wait()
        cn.wait()

        def row_body(r, acc):
            p0 = pos_v[j, r, pl.ds(0, 16)]
            p1 = pos_v[j, r, pl.ds(16, 16)]
            p2 = pos_v[j, r, pl.ds(32, 16)]
            p3 = pos_v[j, r, pl.ds(48, 16)]
            n0 = neg_v[j, r, pl.ds(0, 16)]
            n1 = neg_v[j, r, pl.ds(16, 16)]
            n2 = neg_v[j, r, pl.ds(32, 16)]
            n3 = neg_v[j, r, pl.ds(48, 16)]
            s = (p0 - n0) * u0
            s = s + (p1 - n1) * u1
            s = s + (p2 - n2) * u2
            s = s + (p3 - n3) * u3
            for p in perms:
                s = s + _permute(s, p)
            loss = jnp.maximum(0.0, _MARGIN - s)
            return acc + jnp.where(lane15, loss, 0.0)

        acc = lax.fori_loop(0, _CHUNK, row_body, acc)

    acc_v[...] = acc
    pltpu.sync_copy(acc_v, out_hbm.at[wid])


@jax.jit
def _sc_loss_partials(idx2d, pos, neg, u):
    mesh = plsc.VectorSubcoreMesh(core_axis_name="c", subcore_axis_name="s")
    f = pl.kernel(
        _sc_body,
        out_type=jax.ShapeDtypeStruct((_NW, 16), jnp.float32),
        mesh=mesh,
        scratch_types=[
            pltpu.VMEM((_NCHUNK, _CHUNK), jnp.int32),
            pltpu.VMEM((_NCHUNK, _CHUNK, _D), jnp.float32),
            pltpu.VMEM((_NCHUNK, _CHUNK, _D), jnp.float32),
            pltpu.VMEM((1, _D), jnp.float32),
            pltpu.VMEM((16,), jnp.float32),
            pltpu.SemaphoreType.DMA,
            pltpu.SemaphoreType.DMA,
        ],
        compiler_params=pltpu.CompilerParams(use_tc_tiling_on_sc=False),
    )
    return f(idx2d, pos, neg, u)


def kernel(idxs, positive_samples, negative_samples, U):
    idx2d = idxs.reshape(_NW * _NCHUNK, _CHUNK).astype(jnp.int32)
    partials = _sc_loss_partials(idx2d, positive_samples, negative_samples, U)
    return jnp.sum(partials) / _BATCH


# speedup vs baseline: 1.2635x; 1.0120x over previous
import functools

import jax
import jax.numpy as jnp
from jax import lax
from jax.experimental import pallas as pl
from jax.experimental.pallas import tpu as pltpu
from jax.experimental.pallas import tpu_sc as plsc

_GATHER_DN = lax.GatherDimensionNumbers(
    offset_dims=(), collapsed_slice_dims=(0,), start_index_map=(0,))


def _permute(v, idx):
    return lax.gather(v, idx[:, None], _GATHER_DN, slice_sizes=(1,),
                      mode=lax.GatherScatterMode.PROMISE_IN_BOUNDS)


_BATCH = 16384
_D = 64
_MARGIN = 10.0
_NC = 2
_NS = 16
_NW = _NC * _NS
_ROWS_PER_W = _BATCH // _NW
_CHUNK = 128
_NCHUNK = _ROWS_PER_W // _CHUNK


def _sc_body(idx_hbm, pos_hbm, neg_hbm, u_hbm, out_hbm,
             idx_v, pos_v, neg_v, u_v, acc_v, sem0, sem1, sem2, sem3):
    sems = [sem0, sem1, sem2, sem3]
    wid = lax.axis_index("s") * _NC + lax.axis_index("c")

    pltpu.sync_copy(idx_hbm.at[pl.ds(wid * _NCHUNK, _NCHUNK)], idx_v)
    pltpu.sync_copy(u_hbm, u_v)

    descs = []
    for j in range(_NCHUNK):
        dst = pl.ds(j * _CHUNK, _CHUNK)
        descs.append(pltpu.async_copy(pos_hbm.at[idx_v.at[j]],
                                      pos_v.at[dst], sems[j]))
        descs.append(pltpu.async_copy(neg_hbm.at[idx_v.at[j]],
                                      neg_v.at[dst], sems[j]))

    u0 = u_v[0, pl.ds(0, 16)]
    u1 = u_v[0, pl.ds(16, 16)]
    u2 = u_v[0, pl.ds(32, 16)]
    u3 = u_v[0, pl.ds(48, 16)]
    lanes = lax.iota(jnp.int32, 16)
    lane15 = lanes == 15
    perms = [lanes ^ (1 << k) for k in range(4)]

    def row_body(r, acc):
        p0 = pos_v[r, pl.ds(0, 16)]
        p1 = pos_v[r, pl.ds(16, 16)]
        p2 = pos_v[r, pl.ds(32, 16)]
        p3 = pos_v[r, pl.ds(48, 16)]
        n0 = neg_v[r, pl.ds(0, 16)]
        n1 = neg_v[r, pl.ds(16, 16)]
        n2 = neg_v[r, pl.ds(32, 16)]
        n3 = neg_v[r, pl.ds(48, 16)]
        s = (p0 - n0) * u0
        s = s + (p1 - n1) * u1
        s = s + (p2 - n2) * u2
        s = s + (p3 - n3) * u3
        for p in perms:
            s = s + _permute(s, p)
        loss = jnp.maximum(0.0, _MARGIN - s)
        return acc + jnp.where(lane15, loss, 0.0)

    acc = jnp.zeros((16,), jnp.float32)
    for j in range(_NCHUNK):
        descs[2 * j].wait()
        descs[2 * j + 1].wait()
        acc = plsc.parallel_loop(j * _CHUNK, (j + 1) * _CHUNK,
                                 unroll=8, carry=acc)(row_body)

    acc_v[...] = acc
    pltpu.sync_copy(acc_v, out_hbm.at[wid])


@jax.jit
def _sc_loss_partials(idx2d, pos, neg, u):
    mesh = plsc.VectorSubcoreMesh(core_axis_name="c", subcore_axis_name="s")
    f = pl.kernel(
        _sc_body,
        out_type=jax.ShapeDtypeStruct((_NW, 16), jnp.float32),
        mesh=mesh,
        scratch_types=[
            pltpu.VMEM((_NCHUNK, _CHUNK), jnp.int32),
            pltpu.VMEM((_ROWS_PER_W, _D), jnp.float32),
            pltpu.VMEM((_ROWS_PER_W, _D), jnp.float32),
            pltpu.VMEM((1, _D), jnp.float32),
            pltpu.VMEM((16,), jnp.float32),
            pltpu.SemaphoreType.DMA,
            pltpu.SemaphoreType.DMA,
            pltpu.SemaphoreType.DMA,
            pltpu.SemaphoreType.DMA,
        ],
        compiler_params=pltpu.CompilerParams(use_tc_tiling_on_sc=False),
    )
    return f(idx2d, pos, neg, u)


def kernel(idxs, positive_samples, negative_samples, U):
    idx2d = idxs.reshape(_NW * _NCHUNK, _CHUNK).astype(jnp.int32)
    partials = _sc_loss_partials(idx2d, positive_samples, negative_samples, U)
    return jnp.sum(partials) / _BATCH
